# baseline (speedup 1.0000x reference)
import functools

import jax
import jax.numpy as jnp
from jax import lax
from jax.experimental import pallas as pl
from jax.experimental.pallas import tpu as pltpu
from jax.experimental.pallas import tpu_sc as plsc

N_ROWS = 100000
D = 128
SEGS = 256
NC, NS, L = 2, 16, 16
NW = NC * NS
G = 200
NG = N_ROWS // G
NSLOT = -(-NG // NW)
GA, GB = 128, G - 128
SEG_PER_TILE = SEGS // NS
NBUF = 4


def _sc_partials(featg, ids1d):
    mesh = plsc.VectorSubcoreMesh(
        core_axis_name="c", subcore_axis_name="s", num_cores=NC, num_subcores=NS
    )

    @functools.partial(
        pl.kernel,
        out_type=(
            jax.ShapeDtypeStruct((NC, SEGS, D), jnp.float32),
            jax.ShapeDtypeStruct((NC, NS, SEGS // L, L), jnp.float32),
        ),
        mesh=mesh,
        compiler_params=pltpu.CompilerParams(needs_layout_passes=False),
        scratch_types=[
            pltpu.VMEM((SEG_PER_TILE, D), jnp.float32),
            pltpu.VMEM((SEGS // L + 1, L), jnp.float32),
            pltpu.VMEM_SHARED((SEGS + 1, D), jnp.float32),
            [pltpu.VMEM((G, D), jnp.float32) for _ in range(NBUF)],
            [pltpu.VMEM((GA,), jnp.int32) for _ in range(NBUF)],
            [pltpu.VMEM((GB,), jnp.int32) for _ in range(NBUF)],
            [pltpu.SemaphoreType.DMA for _ in range(NBUF)],
            [pltpu.SemaphoreType.DMA for _ in range(NBUF)],
        ],
    )
    def k(feat_hbm, ids_hbm, psum_hbm, pcnt_hbm,
          zrow_v, cnt_v, acc_sh, rows_bufs, idxa_bufs, idxb_bufs, lsems, ssems):
        cid = lax.axis_index("c")
        sid = lax.axis_index("s")
        wid = sid * NC + cid

        one16 = jnp.full((L,), 1.0, dtype=jnp.float32)
        zero16 = jnp.zeros((L,), dtype=jnp.float32)
        trash16 = jnp.full((L,), SEGS, dtype=jnp.int32)
        for r in range(SEG_PER_TILE):
            for q in range(D // L):
                zrow_v[r, pl.ds(q * L, L)] = zero16
        for q in range(SEGS // L + 1):
            cnt_v[q, :] = zero16

        pltpu.sync_copy(zrow_v, acc_sh.at[pl.ds(sid * SEG_PER_TILE, SEG_PER_TILE)])
        plsc.subcore_barrier()

        def slot_group(t):
            jg = wid + NW * t
            valid = jg < NG
            return jnp.where(valid, jg, 0), valid

        def issue_loads(t, b):
            jg, _ = slot_group(t)
            la = pltpu.async_copy(feat_hbm.at[jg], rows_bufs[b], lsems[b])
            lb = pltpu.async_copy(ids_hbm.at[pl.ds(jg * G, GA)], idxa_bufs[b],
                                  lsems[b])
            lc = pltpu.async_copy(ids_hbm.at[pl.ds(jg * G + GA, GB)], idxb_bufs[b],
                                  lsems[b])
            return (la, lb, lc)

        load_d = [None] * NBUF
        scat_d = [None] * NBUF
        for b in range(NBUF - 1):
            load_d[b] = issue_loads(b, b)

        for t in range(NSLOT):
            cur = t % NBUF
            for d0 in load_d[cur]:
                d0.wait()
            _, valid = slot_group(t)
            for q in range(G // L):
                if q * L < GA:
                    ref, off = idxa_bufs[cur], q * L
                else:
                    ref, off = idxb_bufs[cur], q * L - GA
                iv = ref[pl.ds(off, L)]
                iv = jnp.where(valid, iv, trash16)
                ref[pl.ds(off, L)] = iv
                plsc.addupdate_scatter(
                    cnt_v, [lax.shift_right_logical(iv, 4), lax.bitwise_and(iv, 15)],
                    one16)
            if G % L:
                iv = idxb_bufs[cur][pl.ds(GB - L, L)]
                iv = jnp.where(valid, iv, trash16)
                idxb_bufs[cur][pl.ds(GB - L, L)] = iv
                plsc.addupdate_scatter(
                    cnt_v, [lax.shift_right_logical(iv, 4), lax.bitwise_and(iv, 15)],
                    one16, mask=lax.iota(jnp.int32, L) >= (L - G % L))
            scat_d[cur] = (
                pltpu.async_copy(rows_bufs[cur].at[pl.ds(0, GA)],
                                 acc_sh.at[idxa_bufs[cur]], ssems[cur], add=True),
                pltpu.async_copy(rows_bufs[cur].at[pl.ds(GA, GB)],
                                 acc_sh.at[idxb_bufs[cur]], ssems[cur], add=True),
            )
            jn = t + NBUF - 1
            if jn < NSLOT:
                nxt = jn % NBUF
                if scat_d[nxt] is not None:
                    for d0 in scat_d[nxt]:
                        d0.wait()
                    scat_d[nxt] = None
                load_d[nxt] = issue_loads(jn, nxt)

        for b in range(NBUF):
            if scat_d[b] is not None:
                for d0 in scat_d[b]:
                    d0.wait()
        plsc.subcore_barrier()

        s0 = sid * SEG_PER_TILE
        pltpu.sync_copy(acc_sh.at[pl.ds(s0, SEG_PER_TILE)],
                        psum_hbm.at[cid, pl.ds(s0, SEG_PER_TILE)])
        pltpu.sync_copy(cnt_v.at[pl.ds(0, SEGS // L)], pcnt_hbm.at[cid, sid])

    return k(featg, ids1d)


def _combine_body(psum_ref, pcnt_ref, out_ref):
    s = psum_ref[0] + psum_ref[1]
    c = jnp.sum(pcnt_ref[...], axis=0)
    out_ref[...] = s / jnp.maximum(c, 1.0)[:, None]


def kernel(feat, segment_ids):
    featg = feat.reshape(NG, G, D)
    ids1d = segment_ids.astype(jnp.int32)
    psum, pcnt = _sc_partials(featg, ids1d)
    pcnt2d = pcnt.reshape(NW, SEGS)
    return pl.pallas_call(
        _combine_body,
        out_shape=jax.ShapeDtypeStruct((SEGS, D), jnp.float32),
    )(psum, pcnt2d)


# speedup vs baseline: 1.0064x; 1.0064x over previous
import functools

import jax
import jax.numpy as jnp
from jax import lax
from jax.experimental import pallas as pl
from jax.experimental.pallas import tpu as pltpu
from jax.experimental.pallas import tpu_sc as plsc

N_ROWS = 100000
D = 128
SEGS = 256
NC, NS, L = 2, 16, 16
NW = NC * NS
G = 200
NG = N_ROWS // G
NSLOT = -(-NG // NW)
GA, GB = 128, G - 128
SEG_PER_TILE = SEGS // NS
NBUF = 4


def _sc_partials(featg, ids1d):
    mesh = plsc.VectorSubcoreMesh(
        core_axis_name="c", subcore_axis_name="s", num_cores=NC, num_subcores=NS
    )

    @functools.partial(
        pl.kernel,
        out_type=(
            jax.ShapeDtypeStruct((NC, SEGS, D), jnp.float32),
            jax.ShapeDtypeStruct((NC, NS, SEGS // L, L), jnp.float32),
        ),
        mesh=mesh,
        compiler_params=pltpu.CompilerParams(needs_layout_passes=False),
        scratch_types=[
            pltpu.VMEM((SEG_PER_TILE, D), jnp.float32),
            pltpu.VMEM((SEGS // L + 1, L), jnp.float32),
            pltpu.VMEM_SHARED((SEGS + 1, D), jnp.float32),
            [pltpu.VMEM((G, D), jnp.float32) for _ in range(NBUF)],
            [pltpu.VMEM((GA,), jnp.int32) for _ in range(NBUF)],
            [pltpu.VMEM((GB,), jnp.int32) for _ in range(NBUF)],
            [pltpu.SemaphoreType.DMA for _ in range(NBUF)],
            [pltpu.SemaphoreType.DMA for _ in range(NBUF)],
        ],
    )
    def k(feat_hbm, ids_hbm, psum_hbm, pcnt_hbm,
          zrow_v, cnt_v, acc_sh, rows_bufs, idxa_bufs, idxb_bufs, lsems, ssems):
        cid = lax.axis_index("c")
        sid = lax.axis_index("s")
        wid = sid * NC + cid

        one16 = jnp.full((L,), 1.0, dtype=jnp.float32)
        zero16 = jnp.zeros((L,), dtype=jnp.float32)
        trash16 = jnp.full((L,), SEGS, dtype=jnp.int32)
        for r in range(SEG_PER_TILE):
            for q in range(D // L):
                zrow_v[r, pl.ds(q * L, L)] = zero16
        for q in range(SEGS // L + 1):
            cnt_v[q, :] = zero16

        pltpu.sync_copy(zrow_v, acc_sh.at[pl.ds(sid * SEG_PER_TILE, SEG_PER_TILE)])
        plsc.subcore_barrier()

        def slot_group(t):
            jg = wid + NW * t
            valid = jg < NG
            return jnp.where(valid, jg, 0), valid

        def issue_loads(t, b):
            jg, _ = slot_group(t)
            la = pltpu.async_copy(feat_hbm.at[jg], rows_bufs[b], lsems[b])
            lb = pltpu.async_copy(ids_hbm.at[pl.ds(jg * G, GA)], idxa_bufs[b],
                                  lsems[b])
            lc = pltpu.async_copy(ids_hbm.at[pl.ds(jg * G + GA, GB)], idxb_bufs[b],
                                  lsems[b])
            return (la, lb, lc)

        load_d = [None] * NBUF
        scat_d = [None] * NBUF
        for b in range(NBUF - 1):
            load_d[b] = issue_loads(b, b)

        assert NW * (NSLOT - 1) <= NG

        def hist_vreg(iv, mask=None):
            plsc.addupdate_scatter(
                cnt_v, [lax.shift_right_logical(iv, 4), lax.bitwise_and(iv, 15)],
                one16, mask=mask)

        def idx_vregs(cur):
            out = [(idxa_bufs[cur] if q * L < GA else idxb_bufs[cur],
                    q * L if q * L < GA else q * L - GA, 0)
                   for q in range(G // L)]
            if G % L:
                out.append((idxb_bufs[cur], GB - L, L - G % L))
            return out

        for t in range(NSLOT):
            cur = t % NBUF
            for d0 in load_d[cur]:
                d0.wait()
            last = t == NSLOT - 1
            if last:
                _, valid = slot_group(t)
                for ref, off, skip in idx_vregs(cur):
                    iv = jnp.where(valid, ref[pl.ds(off, L)], trash16)
                    ref[pl.ds(off, L)] = iv
                    hist_vreg(iv, None if not skip
                              else lax.iota(jnp.int32, L) >= skip)
            scat_d[cur] = (
                pltpu.async_copy(rows_bufs[cur].at[pl.ds(0, GA)],
                                 acc_sh.at[idxa_bufs[cur]], ssems[cur], add=True),
                pltpu.async_copy(rows_bufs[cur].at[pl.ds(GA, GB)],
                                 acc_sh.at[idxb_bufs[cur]], ssems[cur], add=True),
            )
            jn = t + NBUF - 1
            if jn < NSLOT:
                nxt = jn % NBUF
                if scat_d[nxt] is not None:
                    for d0 in scat_d[nxt]:
                        d0.wait()
                    scat_d[nxt] = None
                load_d[nxt] = issue_loads(jn, nxt)
            if not last:
                for ref, off, skip in idx_vregs(cur):
                    hist_vreg(ref[pl.ds(off, L)],
                              None if not skip
                              else lax.iota(jnp.int32, L) >= skip)

        for b in range(NBUF):
            if scat_d[b] is not None:
                for d0 in scat_d[b]:
                    d0.wait()
        plsc.subcore_barrier()

        s0 = sid * SEG_PER_TILE
        pltpu.sync_copy(acc_sh.at[pl.ds(s0, SEG_PER_TILE)],
                        psum_hbm.at[cid, pl.ds(s0, SEG_PER_TILE)])
        pltpu.sync_copy(cnt_v.at[pl.ds(0, SEGS // L)], pcnt_hbm.at[cid, sid])

    return k(featg, ids1d)


def _combine_body(psum_ref, pcnt_ref, out_ref):
    s = psum_ref[0] + psum_ref[1]
    c = jnp.sum(pcnt_ref[...], axis=0)
    out_ref[...] = s / jnp.maximum(c, 1.0)[:, None]


def kernel(feat, segment_ids):
    featg = feat.reshape(NG, G, D)
    ids1d = segment_ids.astype(jnp.int32)
    psum, pcnt = _sc_partials(featg, ids1d)
    pcnt2d = pcnt.reshape(NW, SEGS)
    return pl.pallas_call(
        _combine_body,
        out_shape=jax.ShapeDtypeStruct((SEGS, D), jnp.float32),
    )(psum, pcnt2d)


# speedup vs baseline: 1.0183x; 1.0119x over previous
import functools

import jax
import jax.numpy as jnp
from jax import lax
from jax.experimental import pallas as pl
from jax.experimental.pallas import tpu as pltpu
from jax.experimental.pallas import tpu_sc as plsc

N_ROWS = 100000
D = 128
SEGS = 256
NC, NS, L = 2, 16, 16
NW = NC * NS
G = 160
NG = N_ROWS // G
NSLOT = -(-NG // NW)
GA, GB = 128, G - 128
SEG_PER_TILE = SEGS // NS
NBUF = 5


def _sc_partials(featg, ids1d):
    mesh = plsc.VectorSubcoreMesh(
        core_axis_name="c", subcore_axis_name="s", num_cores=NC, num_subcores=NS
    )

    @functools.partial(
        pl.kernel,
        out_type=(
            jax.ShapeDtypeStruct((NC, SEGS, D), jnp.float32),
            jax.ShapeDtypeStruct((NC, NS, SEGS // L, L), jnp.float32),
        ),
        mesh=mesh,
        compiler_params=pltpu.CompilerParams(needs_layout_passes=False),
        scratch_types=[
            pltpu.VMEM((SEG_PER_TILE, D), jnp.float32),
            pltpu.VMEM((SEGS // L + 1, L), jnp.float32),
            pltpu.VMEM_SHARED((SEGS + 1, D), jnp.float32),
            [pltpu.VMEM((G, D), jnp.float32) for _ in range(NBUF)],
            [pltpu.VMEM((GA,), jnp.int32) for _ in range(NBUF)],
            [pltpu.VMEM((GB,), jnp.int32) for _ in range(NBUF)],
            [pltpu.SemaphoreType.DMA for _ in range(NBUF)],
            [pltpu.SemaphoreType.DMA for _ in range(NBUF)],
        ],
    )
    def k(feat_hbm, ids_hbm, psum_hbm, pcnt_hbm,
          zrow_v, cnt_v, acc_sh, rows_bufs, idxa_bufs, idxb_bufs, lsems, ssems):
        cid = lax.axis_index("c")
        sid = lax.axis_index("s")
        wid = sid * NC + cid

        one16 = jnp.full((L,), 1.0, dtype=jnp.float32)
        zero16 = jnp.zeros((L,), dtype=jnp.float32)
        trash16 = jnp.full((L,), SEGS, dtype=jnp.int32)
        for r in range(SEG_PER_TILE):
            for q in range(D // L):
                zrow_v[r, pl.ds(q * L, L)] = zero16
        for q in range(SEGS // L + 1):
            cnt_v[q, :] = zero16

        pltpu.sync_copy(zrow_v, acc_sh.at[pl.ds(sid * SEG_PER_TILE, SEG_PER_TILE)])
        plsc.subcore_barrier()

        def slot_group(t):
            jg = wid + NW * t
            valid = jg < NG
            return jnp.where(valid, jg, 0), valid

        def issue_loads(t, b):
            jg, _ = slot_group(t)
            la = pltpu.async_copy(feat_hbm.at[jg], rows_bufs[b], lsems[b])
            lb = pltpu.async_copy(ids_hbm.at[pl.ds(jg * G, GA)], idxa_bufs[b],
                                  lsems[b])
            lc = pltpu.async_copy(ids_hbm.at[pl.ds(jg * G + GA, GB)], idxb_bufs[b],
                                  lsems[b])
            return (la, lb, lc)

        load_d = [None] * NBUF
        scat_d = [None] * NBUF
        for b in range(NBUF - 1):
            load_d[b] = issue_loads(b, b)

        assert NW * (NSLOT - 1) <= NG

        def hist_vreg(iv, mask=None):
            plsc.addupdate_scatter(
                cnt_v, [lax.shift_right_logical(iv, 4), lax.bitwise_and(iv, 15)],
                one16, mask=mask)

        def idx_vregs(cur):
            out = [(idxa_bufs[cur] if q * L < GA else idxb_bufs[cur],
                    q * L if q * L < GA else q * L - GA, 0)
                   for q in range(G // L)]
            if G % L:
                out.append((idxb_bufs[cur], GB - L, L - G % L))
            return out

        for t in range(NSLOT):
            cur = t % NBUF
            for d0 in load_d[cur]:
                d0.wait()
            last = t == NSLOT - 1
            if last:
                _, valid = slot_group(t)
                for ref, off, skip in idx_vregs(cur):
                    iv = jnp.where(valid, ref[pl.ds(off, L)], trash16)
                    ref[pl.ds(off, L)] = iv
                    hist_vreg(iv, None if not skip
                              else lax.iota(jnp.int32, L) >= skip)
            scat_d[cur] = (
                pltpu.async_copy(rows_bufs[cur].at[pl.ds(0, GA)],
                                 acc_sh.at[idxa_bufs[cur]], ssems[cur], add=True),
                pltpu.async_copy(rows_bufs[cur].at[pl.ds(GA, GB)],
                                 acc_sh.at[idxb_bufs[cur]], ssems[cur], add=True),
            )
            jn = t + NBUF - 1
            if jn < NSLOT:
                nxt = jn % NBUF
                if scat_d[nxt] is not None:
                    for d0 in scat_d[nxt]:
                        d0.wait()
                    scat_d[nxt] = None
                load_d[nxt] = issue_loads(jn, nxt)
            if not last:
                for ref, off, skip in idx_vregs(cur):
                    hist_vreg(ref[pl.ds(off, L)],
                              None if not skip
                              else lax.iota(jnp.int32, L) >= skip)

        for b in range(NBUF):
            if scat_d[b] is not None:
                for d0 in scat_d[b]:
                    d0.wait()
        plsc.subcore_barrier()

        s0 = sid * SEG_PER_TILE
        pltpu.sync_copy(acc_sh.at[pl.ds(s0, SEG_PER_TILE)],
                        psum_hbm.at[cid, pl.ds(s0, SEG_PER_TILE)])
        pltpu.sync_copy(cnt_v.at[pl.ds(0, SEGS // L)], pcnt_hbm.at[cid, sid])

    return k(featg, ids1d)


def _combine_body(psum_ref, pcnt_ref, out_ref):
    s = psum_ref[0] + psum_ref[1]
    c = jnp.sum(pcnt_ref[...], axis=0)
    out_ref[...] = s / jnp.maximum(c, 1.0)[:, None]


def kernel(feat, segment_ids):
    featg = feat.reshape(NG, G, D)
    ids1d = segment_ids.astype(jnp.int32)
    psum, pcnt = _sc_partials(featg, ids1d)
    pcnt2d = pcnt.reshape(NW, SEGS)
    return pl.pallas_call(
        _combine_body,
        out_shape=jax.ShapeDtypeStruct((SEGS, D), jnp.float32),
    )(psum, pcnt2d)
